# baseline (speedup 1.0000x reference)
import functools

import jax
import jax.numpy as jnp
from jax.experimental import pallas as pl
from jax.experimental.pallas import tpu as pltpu


_KB = 8


def _adapter_kernel(x_ref, w1k_ref, w2k_ref, o_ref, *, res_scale, hb):
    _, c, _, w = x_ref.shape
    w1k = w1k_ref[...]
    w2k = w2k_ref[...]
    for g in range(hb // (2 * _KB)):
        k0, k1 = 2 * g, 2 * g + 1
        xa = x_ref[0, :, k0 * _KB:(k0 + 1) * _KB, :].reshape(c * _KB, w)
        xc = x_ref[0, :, k1 * _KB:(k1 + 1) * _KB, :].reshape(c * _KB, w)
        x2 = jnp.concatenate([xa, xc], axis=1)
        xb = x2.astype(jnp.bfloat16)
        z = jnp.dot(w1k, xb, preferred_element_type=jnp.float32)
        zb = jnp.maximum(z, 0.0).astype(jnp.bfloat16)
        y = jnp.dot(w2k, zb, preferred_element_type=jnp.float32)
        o = jnp.maximum(y, 0.0) + res_scale * x2
        o_ref[0, :, k0 * _KB:(k0 + 1) * _KB, :] = (
            o[:, :w].reshape(c, _KB, w).astype(o_ref.dtype))
        o_ref[0, :, k1 * _KB:(k1 + 1) * _KB, :] = (
            o[:, w:].reshape(c, _KB, w).astype(o_ref.dtype))


def _kron_eye(w, kb):
    m, n = w.shape
    wf = w.astype(jnp.float32)
    wr = jnp.repeat(wf, kb, axis=0)
    src = jax.lax.broadcasted_iota(jnp.int32, (n, n * kb), 0)
    dst = jax.lax.broadcasted_iota(jnp.int32, (n, n * kb), 1)
    expand = (src == dst // kb).astype(jnp.float32)
    wk = jnp.dot(wr, expand)
    ri = jax.lax.broadcasted_iota(jnp.int32, (m * kb, n * kb), 0)
    ci = jax.lax.broadcasted_iota(jnp.int32, (m * kb, n * kb), 1)
    return jnp.where(ri % kb == ci % kb, wk, 0.0).astype(jnp.bfloat16)


def _token_kernel(x_ref, w1t_ref, w2t_ref, o_ref, *, res_scale):
    x = x_ref[0]
    xb = x.astype(jnp.bfloat16)
    z = jnp.dot(xb, w1t_ref[...], preferred_element_type=jnp.float32)
    zb = jnp.maximum(z, 0.0).astype(jnp.bfloat16)
    y = jnp.dot(zb, w2t_ref[...], preferred_element_type=jnp.float32)
    y = jnp.maximum(y, 0.0)
    o_ref[0] = (y + res_scale * x).astype(o_ref.dtype)


def _token_pair_kernel(xc_ref, xl_ref, w1c_ref, w2c_ref, w1l_ref, w2l_ref,
                       oc_ref, ol_ref, *, rs_c, rs_l):
    j = pl.program_id(1)

    @pl.when(j == 0)
    def _():
        x = xc_ref[0]
        xb = x.astype(jnp.bfloat16)
        z = jnp.dot(xb, w1c_ref[...], preferred_element_type=jnp.float32)
        zb = jnp.maximum(z, 0.0).astype(jnp.bfloat16)
        y = jnp.dot(zb, w2c_ref[...], preferred_element_type=jnp.float32)
        oc_ref[0] = (jnp.maximum(y, 0.0) + rs_c * x).astype(oc_ref.dtype)

    @pl.when(j == 1)
    def _():
        x = xl_ref[0]
        xb = x.astype(jnp.bfloat16)
        z = jnp.dot(xb, w1l_ref[...], preferred_element_type=jnp.float32)
        zb = jnp.maximum(z, 0.0).astype(jnp.bfloat16)
        y = jnp.dot(zb, w2l_ref[...], preferred_element_type=jnp.float32)
        ol_ref[0] = (jnp.maximum(y, 0.0) + rs_l * x).astype(ol_ref.dtype)


def _wt_tok(w1, w2, ratio):
    w1t = jnp.transpose(w1).astype(jnp.bfloat16)
    w2t = (jnp.transpose(w2).astype(jnp.float32)
           * jnp.float32(ratio)).astype(jnp.bfloat16)
    return w1t, w2t


def _adapt_pair_tok(featc, featl, w1c, w2c, w1l, w2l, r_c, r_l):
    bs, c, H, W = featc.shape
    hw = H * W
    xc = jnp.transpose(featc, (0, 2, 3, 1)).reshape(bs, hw, c)
    xl = jnp.transpose(featl, (0, 2, 3, 1)).reshape(bs, hw, c)
    w1ct, w2ct = _wt_tok(w1c, w2c, r_c)
    w1lt, w2lt = _wt_tok(w1l, w2l, r_l)

    io_spec = pl.BlockSpec((1, hw, c), lambda b, j: (b, 0, 0))
    wspec = lambda w: pl.BlockSpec(w.shape, lambda b, j: (0, 0))
    outc, outl = pl.pallas_call(
        functools.partial(_token_pair_kernel, rs_c=1.0 - float(r_c),
                          rs_l=1.0 - float(r_l)),
        out_shape=(jax.ShapeDtypeStruct((bs, hw, c), featc.dtype),
                   jax.ShapeDtypeStruct((bs, hw, c), featl.dtype)),
        grid=(bs, 2),
        in_specs=[io_spec, io_spec,
                  wspec(w1ct), wspec(w2ct), wspec(w1lt), wspec(w2lt)],
        out_specs=(io_spec, io_spec),
        compiler_params=pltpu.CompilerParams(
            dimension_semantics=("parallel", "arbitrary"),
        ),
    )(xc, xl, w1ct, w2ct, w1lt, w2lt)

    def back(o):
        return jnp.transpose(o.reshape(bs, H, W, c), (0, 3, 1, 2))
    return back(outc), back(outl)


def _adapt_one(feat, w1, w2, ratio, *, hb=128):
    bs, c, H, W = feat.shape
    out_shape = feat.shape
    hb = min(hb, H)
    assert H % hb == 0 and hb % _KB == 0, (H, hb)
    w1k = _kron_eye(w1, _KB)
    w2k = _kron_eye(w2.astype(jnp.float32) * jnp.float32(ratio), _KB)

    out = pl.pallas_call(
        functools.partial(_adapter_kernel, res_scale=1.0 - float(ratio),
                          hb=hb),
        out_shape=jax.ShapeDtypeStruct((bs, c, H, W), feat.dtype),
        grid=(bs, H // hb),
        in_specs=[
            pl.BlockSpec((1, c, hb, W), lambda b, j: (b, 0, j, 0)),
            pl.BlockSpec(w1k.shape, lambda b, j: (0, 0)),
            pl.BlockSpec(w2k.shape, lambda b, j: (0, 0)),
        ],
        out_specs=pl.BlockSpec((1, c, hb, W), lambda b, j: (b, 0, j, 0)),
        compiler_params=pltpu.CompilerParams(
            dimension_semantics=("parallel", "parallel"),
        ),
    )(feat, w1k, w2k)
    return out.reshape(out_shape)


def kernel(src_p3_camera, src_p3_lidar, src_p4_camera, src_p4_lidar,
           src_p5_camera, src_p5_lidar,
           w1_p3_camera, w2_p3_camera, w1_p3_lidar, w2_p3_lidar,
           w1_p4_camera, w2_p4_camera, w1_p4_lidar, w2_p4_lidar):
    r_cam, r_lid = 0.2, 0.6
    p4_cam, p4_lid = _adapt_pair_tok(src_p4_camera, src_p4_lidar,
                                     w1_p4_camera, w2_p4_camera,
                                     w1_p4_lidar, w2_p4_lidar, r_cam, r_lid)
    return {
        "p3": {
            "camera": _adapt_one(src_p3_camera, w1_p3_camera, w2_p3_camera,
                                 r_cam),
            "lidar": _adapt_one(src_p3_lidar, w1_p3_lidar, w2_p3_lidar, r_lid),
        },
        "p4": {"camera": p4_cam, "lidar": p4_lid},
        "p5": {"camera": src_p5_camera, "lidar": src_p5_lidar},
    }


# speedup vs baseline: 1.0314x; 1.0314x over previous
import functools

import jax
import jax.numpy as jnp
from jax.experimental import pallas as pl
from jax.experimental.pallas import tpu as pltpu


_KB = 8


def _adapter_kernel(x_ref, w1k_ref, w2k_ref, o_ref, *, res_scale, hb):
    _, c, _, w = x_ref.shape
    w1k = w1k_ref[...]
    w2k = w2k_ref[...]
    for g in range(hb // (2 * _KB)):
        k0, k1 = 2 * g, 2 * g + 1
        xa = x_ref[0, :, k0 * _KB:(k0 + 1) * _KB, :].reshape(c * _KB, w)
        xc = x_ref[0, :, k1 * _KB:(k1 + 1) * _KB, :].reshape(c * _KB, w)
        x2 = jnp.concatenate([xa, xc], axis=1)
        xb = x2.astype(jnp.bfloat16)
        z = jnp.dot(w1k, xb, preferred_element_type=jnp.float32)
        zb = jnp.maximum(z, 0.0).astype(jnp.bfloat16)
        y = jnp.dot(w2k, zb, preferred_element_type=jnp.float32)
        o = jnp.maximum(y, 0.0) + res_scale * x2
        o_ref[0, :, k0 * _KB:(k0 + 1) * _KB, :] = (
            o[:, :w].reshape(c, _KB, w).astype(o_ref.dtype))
        o_ref[0, :, k1 * _KB:(k1 + 1) * _KB, :] = (
            o[:, w:].reshape(c, _KB, w).astype(o_ref.dtype))


def _kron_eye(w, kb):
    m, n = w.shape
    wf = w.astype(jnp.float32)
    wr = jnp.repeat(wf, kb, axis=0)
    src = jax.lax.broadcasted_iota(jnp.int32, (n, n * kb), 0)
    dst = jax.lax.broadcasted_iota(jnp.int32, (n, n * kb), 1)
    expand = (src == dst // kb).astype(jnp.float32)
    wk = jnp.dot(wr, expand)
    ri = jax.lax.broadcasted_iota(jnp.int32, (m * kb, n * kb), 0)
    ci = jax.lax.broadcasted_iota(jnp.int32, (m * kb, n * kb), 1)
    return jnp.where(ri % kb == ci % kb, wk, 0.0).astype(jnp.bfloat16)


def _token_kernel(x_ref, w1t_ref, w2t_ref, o_ref, *, res_scale):
    x = x_ref[0]
    xb = x.astype(jnp.bfloat16)
    z = jnp.dot(xb, w1t_ref[...], preferred_element_type=jnp.float32)
    zb = jnp.maximum(z, 0.0).astype(jnp.bfloat16)
    y = jnp.dot(zb, w2t_ref[...], preferred_element_type=jnp.float32)
    y = jnp.maximum(y, 0.0)
    o_ref[0] = (y + res_scale * x).astype(o_ref.dtype)


def _adapt_one_tok(feat, w1, w2, ratio, *, tb=4096):
    bs, c, H, W = feat.shape
    hw = H * W
    tb = min(tb, hw)
    assert hw % tb == 0, (hw, tb)
    xt = jnp.transpose(feat, (0, 2, 3, 1)).reshape(bs, hw, c)
    w1t = jnp.transpose(w1).astype(jnp.bfloat16)
    w2t = (jnp.transpose(w2).astype(jnp.float32)
           * jnp.float32(ratio)).astype(jnp.bfloat16)

    out = pl.pallas_call(
        functools.partial(_token_kernel, res_scale=1.0 - float(ratio)),
        out_shape=jax.ShapeDtypeStruct((bs, hw, c), feat.dtype),
        grid=(bs, hw // tb),
        in_specs=[
            pl.BlockSpec((1, tb, c), lambda b, j: (b, j, 0)),
            pl.BlockSpec(w1t.shape, lambda b, j: (0, 0)),
            pl.BlockSpec(w2t.shape, lambda b, j: (0, 0)),
        ],
        out_specs=pl.BlockSpec((1, tb, c), lambda b, j: (b, j, 0)),
        compiler_params=pltpu.CompilerParams(
            dimension_semantics=("parallel", "parallel"),
        ),
    )(xt, w1t, w2t)
    return jnp.transpose(out.reshape(bs, H, W, c), (0, 3, 1, 2))


def _adapt_one(feat, w1, w2, ratio, *, hb=128):
    bs, c, H, W = feat.shape
    out_shape = feat.shape
    hb = min(hb, H)
    assert H % hb == 0 and hb % _KB == 0, (H, hb)
    w1k = _kron_eye(w1, _KB)
    w2k = _kron_eye(w2.astype(jnp.float32) * jnp.float32(ratio), _KB)

    out = pl.pallas_call(
        functools.partial(_adapter_kernel, res_scale=1.0 - float(ratio),
                          hb=hb),
        out_shape=jax.ShapeDtypeStruct((bs, c, H, W), feat.dtype),
        grid=(bs, H // hb),
        in_specs=[
            pl.BlockSpec((1, c, hb, W), lambda b, j: (b, 0, j, 0)),
            pl.BlockSpec(w1k.shape, lambda b, j: (0, 0)),
            pl.BlockSpec(w2k.shape, lambda b, j: (0, 0)),
        ],
        out_specs=pl.BlockSpec((1, c, hb, W), lambda b, j: (b, 0, j, 0)),
        compiler_params=pltpu.CompilerParams(
            dimension_semantics=("parallel", "parallel"),
        ),
    )(feat, w1k, w2k)
    return out.reshape(out_shape)


def kernel(src_p3_camera, src_p3_lidar, src_p4_camera, src_p4_lidar,
           src_p5_camera, src_p5_lidar,
           w1_p3_camera, w2_p3_camera, w1_p3_lidar, w2_p3_lidar,
           w1_p4_camera, w2_p4_camera, w1_p4_lidar, w2_p4_lidar):
    r_cam, r_lid = 0.2, 0.6
    return {
        "p3": {
            "camera": _adapt_one(src_p3_camera, w1_p3_camera, w2_p3_camera,
                                 r_cam),
            "lidar": _adapt_one(src_p3_lidar, w1_p3_lidar, w2_p3_lidar, r_lid),
        },
        "p4": {
            "camera": _adapt_one_tok(src_p4_camera, w1_p4_camera,
                                     w2_p4_camera, r_cam),
            "lidar": _adapt_one_tok(src_p4_lidar, w1_p4_lidar, w2_p4_lidar,
                                    r_lid),
        },
        "p5": {"camera": src_p5_camera, "lidar": src_p5_lidar},
    }
